# baseline (speedup 1.0000x reference)
import functools

import jax
import jax.numpy as jnp
from jax import lax
from jax.experimental import pallas as pl
from jax.experimental.pallas import tpu as pltpu
from jax.experimental.pallas import tpu_sc as plsc

N = 10000
E = 320000
D = 128
NC = 2
NS = 16
L = 16
NW = NC * NS
K = 64
G = 16
NG0 = 9
NG1 = 11
NG = NG0 + NG1
CWD = 80
KD = 128
TOTG = NS * NG
EPAD = TOTG * G * K
NPAD = 10240
RPS = NPAD // NS

def _mk_perm():
    pi = [0] * D
    for m in range(D // 32):
        for i in range(16):
            pi[32 * m + i] = 32 * m + 2 * i
            pi[32 * m + 16 + i] = 32 * m + 2 * i + 1
    inv = [0] * D
    for t, q in enumerate(pi):
        inv[q] = t
    return tuple(inv)

_PERM = _mk_perm()

_mesh = plsc.VectorSubcoreMesh(core_axis_name="c", subcore_axis_name="s")
_sc_params = pltpu.CompilerParams(needs_layout_passes=False,
                                  use_tc_tiling_on_sc=False)


@functools.partial(
    pl.kernel,
    out_type=jax.ShapeDtypeStruct((NC, NPAD), jnp.float32),
    mesh=_mesh,
    compiler_params=_sc_params,
    scratch_types=[
        pltpu.VMEM((CWD, KD), jnp.int32),
        pltpu.VMEM((CWD, KD), jnp.float32),
        pltpu.VMEM_SHARED((NPAD,), jnp.float32),
    ],
)
def _sc_deg(dst_hbm, ew_hbm, zdeg_hbm, deg_out, dst_v, ew_v, deg_sh):
    c = lax.axis_index("c")
    s = lax.axis_index("s")
    wid = s * NC + c
    pltpu.sync_copy(zdeg_hbm.at[pl.ds(s * RPS, RPS)],
                    deg_sh.at[pl.ds(s * RPS, RPS)])
    pltpu.sync_copy(dst_hbm.at[wid], dst_v)
    pltpu.sync_copy(ew_hbm.at[wid], ew_v)
    plsc.subcore_barrier()

    @pl.loop(0, CWD)
    def _(t):
        pltpu.sync_copy(ew_v.at[t], deg_sh.at[dst_v.at[t]], add=True)

    plsc.subcore_barrier()

    @pl.when(s == 0)
    def _():
        pltpu.sync_copy(deg_sh, deg_out.at[c])


@functools.partial(
    pl.kernel,
    out_type=jax.ShapeDtypeStruct((NC, NPAD, D), jnp.float32),
    mesh=_mesh,
    compiler_params=_sc_params,
    scratch_types=[
        pltpu.VMEM((2, G, K), jnp.int32),
        pltpu.VMEM((2, G, K), jnp.int32),
        pltpu.VMEM((2, G * K), jnp.float32),
        pltpu.VMEM((K, D // 2), jnp.int32),
        pltpu.VMEM((K, D // 2), jnp.int32),
        pltpu.VMEM((K, D // 2), jnp.int32),
        pltpu.VMEM((K, D // 2), jnp.int32),
        pltpu.VMEM((K, D), jnp.float32),
        pltpu.VMEM((K, D), jnp.float32),
        pltpu.VMEM_SHARED((NPAD, D), jnp.float32),
        pltpu.SemaphoreType.DMA,
        pltpu.SemaphoreType.DMA,
        pltpu.SemaphoreType.DMA,
        pltpu.SemaphoreType.DMA,
        pltpu.SemaphoreType.DMA,
        pltpu.SemaphoreType.DMA,
        pltpu.SemaphoreType.DMA,
    ],
)
def _sc_edge(g_hbm, src_hbm, dst_hbm, ew_hbm, zrow_hbm, acc_out,
             src_v, dst_v, ew_v, rb0, rb1, rb2, rb3, rc0, rc1, acc_sh,
             sg0, sg1, sg2, sg3, ss0, ss1, si):
    c = lax.axis_index("c")
    s = lax.axis_index("s")
    ng = jnp.where(c == 0, NG0, NG1)
    gb = jnp.where(c == 0, s * NG0, NS * NG0 + s * NG1)
    rb = [rb0, rb1, rb2, rb3]
    rc = [rc0, rc1]
    sg = [sg0, sg1, sg2, sg3]
    ss = [ss0, ss1]

    pltpu.sync_copy(zrow_hbm.at[pl.ds(s * RPS, RPS)],
                    acc_sh.at[pl.ds(s * RPS, RPS)])

    def load_group(gi, p):
        pltpu.make_async_copy(src_hbm.at[gb + gi], src_v.at[p], si).start()
        pltpu.make_async_copy(dst_hbm.at[gb + gi], dst_v.at[p], si).start()
        pltpu.make_async_copy(ew_hbm.at[gb + gi], ew_v.at[p], si).start()

    def wait_group(p):
        pltpu.make_async_copy(src_hbm.at[gb], src_v.at[p], si).wait()
        pltpu.make_async_copy(dst_hbm.at[gb], dst_v.at[p], si).wait()
        pltpu.make_async_copy(ew_hbm.at[gb], ew_v.at[p], si).wait()

    def start_gather(p, t, i):
        pltpu.make_async_copy(g_hbm.at[src_v.at[p, t]], rb[i], sg[i]).start()

    def wait_gather(p, i):
        pltpu.make_async_copy(g_hbm.at[src_v.at[p, 0]], rb[i], sg[i]).wait()

    def start_scatter(p, t, j):
        pltpu.make_async_copy(
            rc[j], acc_sh.at[dst_v.at[p, t]], ss[j]).start(add=True)

    def wait_scatter(p, j):
        pltpu.make_async_copy(rc[j], acc_sh.at[dst_v.at[p, 0]], ss[j]).wait()

    def scale(p, t, i, j):
        @pl.loop(0, K)
        def _(k):
            w = plsc.load_gather(
                ew_v, [jnp.full((L,), p, jnp.int32),
                       jnp.full((L,), t * K + k, jnp.int32)])
            for m in range(D // 32):
                v32 = rb[i][k, pl.ds(16 * m, 16)]
                v = plsc.bitcast(v32, jnp.bfloat16)
                a, b = plsc.unpack(v, format=plsc.PackFormat.INTERLEAVED)
                rc[j][k, pl.ds(32 * m, L)] = a * w
                rc[j][k, pl.ds(32 * m + L, L)] = b * w

    @pl.when(ng > 0)
    def _():
        load_group(0, 0)

    plsc.subcore_barrier()

    @pl.when(ng > 0)
    def _():
        wait_group(0)
        for i in range(4):
            start_gather(0, i, i)

    @pl.loop(0, NG)
    def _(gi):
        @pl.when(gi < ng)
        def _():
            p = lax.rem(gi, 2)

            @pl.loop(0, G // 4)
            def _(q):
                for i in range(4):
                    t = q * 4 + i
                    j = i % 2
                    wait_gather(p, i)
                    if i < 2:
                        @pl.when(jnp.logical_or(gi > 0, q > 0))
                        def _():
                            wait_scatter(p, j)
                    else:
                        wait_scatter(p, j)
                    if i == 0:
                        @pl.when(jnp.logical_and(q == 1, gi + 1 < ng))
                        def _():
                            load_group(gi + 1, 1 - p)
                    scale(p, t, i, j)
                    start_scatter(p, t, j)

                    @pl.when(q < G // 4 - 1)
                    def _():
                        start_gather(p, t + 4, i)

                    @pl.when(jnp.logical_and(q == G // 4 - 1, gi + 1 < ng))
                    def _():
                        if i == 0:
                            wait_group(1 - p)
                        start_gather(1 - p, i, i)

    @pl.when(ng > 0)
    def _():
        wait_scatter(0, 0)
        wait_scatter(0, 1)

    plsc.subcore_barrier()
    pltpu.sync_copy(acc_sh.at[pl.ds(s * RPS, RPS)],
                    acc_out.at[c, pl.ds(s * RPS, RPS)])


def _tc_dinv(deg2):
    def body(deg_ref, out_ref):
        d = deg_ref[0:1, :] + deg_ref[1:2, :] + 1.0
        out_ref[...] = jnp.where(d > 0, lax.rsqrt(d), 0.0)

    return pl.pallas_call(
        body, out_shape=jax.ShapeDtypeStruct((1, NPAD), jnp.float32))(deg2)


def _tc_mm_scale(x, w, ws, dinv):
    def body(x_ref, w_ref, ws_ref, dinv_ref, o_ref, ob_ref):
        h = jnp.dot(x_ref[...], w_ref[...], preferred_element_type=jnp.float32)
        hs = jnp.dot(x_ref[...], ws_ref[...],
                     preferred_element_type=jnp.float32)
        o_ref[...] = h * dinv_ref[...]
        ob_ref[...] = (hs * dinv_ref[...]).astype(jnp.bfloat16)

    return pl.pallas_call(
        body, out_shape=(jax.ShapeDtypeStruct((N, D), jnp.float32),
                         jax.ShapeDtypeStruct((N, D), jnp.bfloat16)))(
                             x, w, ws, dinv)


def _tc_post_mm(acc, g, dinv, b, w, ws):
    def body(acc_ref, g_ref, dinv_ref, b_ref, w_ref, ws_ref, o_ref, ob_ref):
        agg = acc_ref[0, :N, :] + acc_ref[1, :N, :] + g_ref[...]
        x1 = jnp.maximum(dinv_ref[...] * agg + b_ref[...], 0.0)
        o_ref[...] = jnp.dot(
            x1, w_ref[...], preferred_element_type=jnp.float32) * dinv_ref[...]
        ob_ref[...] = (jnp.dot(
            x1, ws_ref[...],
            preferred_element_type=jnp.float32) * dinv_ref[...]).astype(
                jnp.bfloat16)

    return pl.pallas_call(
        body, out_shape=(jax.ShapeDtypeStruct((N, D), jnp.float32),
                         jax.ShapeDtypeStruct((N, D), jnp.bfloat16)))(
                             acc, g, dinv, b, w, ws)


def _tc_post_final(acc, g, dinv, b):
    def body(acc_ref, g_ref, dinv_ref, b_ref, o_ref):
        agg = acc_ref[0, :N, :] + acc_ref[1, :N, :] + g_ref[...]
        o_ref[...] = jnp.maximum(dinv_ref[...] * agg + b_ref[...], 0.0)

    return pl.pallas_call(
        body, out_shape=jax.ShapeDtypeStruct((N, D), jnp.float32))(
            acc, g, dinv, b)


@jax.jit
def kernel(x, edge_index, edge_weight, W1, b1, W2, b2):
    src = edge_index[0].astype(jnp.int32)
    dst = edge_index[1].astype(jnp.int32)
    ew = edge_weight.astype(jnp.float32)
    pad = EPAD - E
    src_p = jnp.pad(src, (0, pad))
    dst_p = jnp.pad(dst, (0, pad))
    ew_p = jnp.pad(ew, (0, pad))
    src4 = src_p.reshape(TOTG, G, K)
    dst4 = dst_p.reshape(TOTG, G, K)
    ew4 = ew_p.reshape(TOTG, G * K)
    dst3 = dst_p.reshape(NW, CWD, KD)
    ew3 = ew_p.reshape(NW, CWD, KD)
    zdeg = jnp.zeros((NPAD,), jnp.float32)
    zrow = jnp.zeros((NPAD, D), jnp.float32)
    perm = jnp.asarray(_PERM, dtype=jnp.int32)
    W1s = jnp.take(W1, perm, axis=1)
    W2s = jnp.take(W2, perm, axis=1)

    def as_words(gb):
        return lax.bitcast_convert_type(
            gb.reshape(N, D // 2, 2), jnp.int32)

    deg2 = _sc_deg(dst3, ew3, zdeg)
    dinv_row = _tc_dinv(deg2)
    dinv_col = dinv_row[0, :N][:, None]

    g1, g1b = _tc_mm_scale(x, W1, W1s, dinv_col)
    acc1 = _sc_edge(as_words(g1b), src4, dst4, ew4, zrow)
    g2, g2b = _tc_post_mm(acc1, g1, dinv_col, b1.reshape(1, D), W2, W2s)
    acc2 = _sc_edge(as_words(g2b), src4, dst4, ew4, zrow)
    return _tc_post_final(acc2, g2, dinv_col, b2.reshape(1, D))


# speedup vs baseline: 1.0706x; 1.0706x over previous
import functools

import jax
import jax.numpy as jnp
from jax import lax
from jax.experimental import pallas as pl
from jax.experimental.pallas import tpu as pltpu
from jax.experimental.pallas import tpu_sc as plsc

N = 10000
E = 320000
D = 128
NC = 2
NS = 16
L = 16
NW = NC * NS
K = 64
G = 16
NG0 = 10
NG1 = 10
NG = NG0 + NG1
CWD = 80
KD = 128
TOTG = NS * NG
EPAD = TOTG * G * K
NPAD = 10240
RPS = NPAD // NS

def _mk_perm():
    pi = [0] * D
    for m in range(D // 32):
        for i in range(16):
            pi[32 * m + i] = 32 * m + 2 * i
            pi[32 * m + 16 + i] = 32 * m + 2 * i + 1
    inv = [0] * D
    for t, q in enumerate(pi):
        inv[q] = t
    return tuple(inv)

_PERM = _mk_perm()

_mesh = plsc.VectorSubcoreMesh(core_axis_name="c", subcore_axis_name="s")
_sc_params = pltpu.CompilerParams(needs_layout_passes=False,
                                  use_tc_tiling_on_sc=False)


@functools.partial(
    pl.kernel,
    out_type=jax.ShapeDtypeStruct((NC, NPAD), jnp.float32),
    mesh=_mesh,
    compiler_params=_sc_params,
    scratch_types=[
        pltpu.VMEM((CWD, KD), jnp.int32),
        pltpu.VMEM((CWD, KD), jnp.float32),
        pltpu.VMEM_SHARED((NPAD,), jnp.float32),
    ],
)
def _sc_deg(dst_hbm, ew_hbm, zdeg_hbm, deg_out, dst_v, ew_v, deg_sh):
    c = lax.axis_index("c")
    s = lax.axis_index("s")
    wid = s * NC + c
    pltpu.sync_copy(zdeg_hbm.at[pl.ds(s * RPS, RPS)],
                    deg_sh.at[pl.ds(s * RPS, RPS)])
    pltpu.sync_copy(dst_hbm.at[wid], dst_v)
    pltpu.sync_copy(ew_hbm.at[wid], ew_v)
    plsc.subcore_barrier()

    @pl.loop(0, CWD)
    def _(t):
        pltpu.sync_copy(ew_v.at[t], deg_sh.at[dst_v.at[t]], add=True)

    plsc.subcore_barrier()

    @pl.when(s == 0)
    def _():
        pltpu.sync_copy(deg_sh, deg_out.at[c])


@functools.partial(
    pl.kernel,
    out_type=jax.ShapeDtypeStruct((NC, NPAD, D), jnp.float32),
    mesh=_mesh,
    compiler_params=_sc_params,
    scratch_types=[
        pltpu.VMEM((2, G, K), jnp.int32),
        pltpu.VMEM((2, G, K), jnp.int32),
        pltpu.VMEM((2, G * K), jnp.float32),
        pltpu.VMEM((K, D // 2), jnp.int32),
        pltpu.VMEM((K, D // 2), jnp.int32),
        pltpu.VMEM((K, D // 2), jnp.int32),
        pltpu.VMEM((K, D // 2), jnp.int32),
        pltpu.VMEM((K, D), jnp.float32),
        pltpu.VMEM((K, D), jnp.float32),
        pltpu.VMEM_SHARED((NPAD, D), jnp.float32),
        pltpu.SemaphoreType.DMA,
        pltpu.SemaphoreType.DMA,
        pltpu.SemaphoreType.DMA,
        pltpu.SemaphoreType.DMA,
        pltpu.SemaphoreType.DMA,
        pltpu.SemaphoreType.DMA,
        pltpu.SemaphoreType.DMA,
    ],
)
def _sc_edge(g_hbm, src_hbm, dst_hbm, ew_hbm, zrow_hbm, acc_out,
             src_v, dst_v, ew_v, rb0, rb1, rb2, rb3, rc0, rc1, acc_sh,
             sg0, sg1, sg2, sg3, ss0, ss1, si):
    c = lax.axis_index("c")
    s = lax.axis_index("s")
    ng = jnp.where(c == 0, NG0, NG1)
    gb = jnp.where(c == 0, s * NG0, NS * NG0 + s * NG1)
    rb = [rb0, rb1, rb2, rb3]
    rc = [rc0, rc1]
    sg = [sg0, sg1, sg2, sg3]
    ss = [ss0, ss1]

    pltpu.sync_copy(zrow_hbm.at[pl.ds(s * RPS, RPS)],
                    acc_sh.at[pl.ds(s * RPS, RPS)])

    def load_group(gi, p):
        pltpu.make_async_copy(src_hbm.at[gb + gi], src_v.at[p], si).start()
        pltpu.make_async_copy(dst_hbm.at[gb + gi], dst_v.at[p], si).start()
        pltpu.make_async_copy(ew_hbm.at[gb + gi], ew_v.at[p], si).start()

    def wait_group(p):
        pltpu.make_async_copy(src_hbm.at[gb], src_v.at[p], si).wait()
        pltpu.make_async_copy(dst_hbm.at[gb], dst_v.at[p], si).wait()
        pltpu.make_async_copy(ew_hbm.at[gb], ew_v.at[p], si).wait()

    def start_gather(p, t, i):
        pltpu.make_async_copy(g_hbm.at[src_v.at[p, t]], rb[i], sg[i]).start()

    def wait_gather(p, i):
        pltpu.make_async_copy(g_hbm.at[src_v.at[p, 0]], rb[i], sg[i]).wait()

    def start_scatter(p, t, j):
        pltpu.make_async_copy(
            rc[j], acc_sh.at[dst_v.at[p, t]], ss[j]).start(add=True)

    def wait_scatter(p, j):
        pltpu.make_async_copy(rc[j], acc_sh.at[dst_v.at[p, 0]], ss[j]).wait()

    def scale(p, t, i, j):
        @pl.loop(0, K)
        def _(k):
            w = plsc.load_gather(
                ew_v, [jnp.full((L,), p, jnp.int32),
                       jnp.full((L,), t * K + k, jnp.int32)])
            for m in range(D // 32):
                v32 = rb[i][k, pl.ds(16 * m, 16)]
                v = plsc.bitcast(v32, jnp.bfloat16)
                a, b = plsc.unpack(v, format=plsc.PackFormat.INTERLEAVED)
                rc[j][k, pl.ds(32 * m, L)] = a * w
                rc[j][k, pl.ds(32 * m + L, L)] = b * w

    @pl.when(ng > 0)
    def _():
        load_group(0, 0)

    plsc.subcore_barrier()

    @pl.when(ng > 0)
    def _():
        wait_group(0)
        for i in range(4):
            start_gather(0, i, i)

    @pl.loop(0, NG)
    def _(gi):
        @pl.when(gi < ng)
        def _():
            p = lax.rem(gi, 2)

            @pl.loop(0, G // 4)
            def _(q):
                for i in range(4):
                    t = q * 4 + i
                    j = i % 2
                    wait_gather(p, i)
                    if i < 2:
                        @pl.when(jnp.logical_or(gi > 0, q > 0))
                        def _():
                            wait_scatter(p, j)
                    else:
                        wait_scatter(p, j)
                    if i == 0:
                        @pl.when(jnp.logical_and(q == 1, gi + 1 < ng))
                        def _():
                            load_group(gi + 1, 1 - p)
                    scale(p, t, i, j)
                    start_scatter(p, t, j)

                    @pl.when(q < G // 4 - 1)
                    def _():
                        start_gather(p, t + 4, i)

                    @pl.when(jnp.logical_and(q == G // 4 - 1, gi + 1 < ng))
                    def _():
                        if i == 0:
                            wait_group(1 - p)
                        start_gather(1 - p, i, i)

    @pl.when(ng > 0)
    def _():
        wait_scatter(0, 0)
        wait_scatter(0, 1)

    plsc.subcore_barrier()
    pltpu.sync_copy(acc_sh.at[pl.ds(s * RPS, RPS)],
                    acc_out.at[c, pl.ds(s * RPS, RPS)])


def _tc_dinv(deg2):
    def body(deg_ref, out_ref):
        d = deg_ref[0:1, :] + deg_ref[1:2, :] + 1.0
        out_ref[...] = jnp.where(d > 0, lax.rsqrt(d), 0.0)

    return pl.pallas_call(
        body, out_shape=jax.ShapeDtypeStruct((1, NPAD), jnp.float32))(deg2)


def _tc_mm_scale(x, w, ws, dinv):
    def body(x_ref, w_ref, ws_ref, dinv_ref, o_ref, ob_ref):
        h = jnp.dot(x_ref[...], w_ref[...], preferred_element_type=jnp.float32)
        hs = jnp.dot(x_ref[...], ws_ref[...],
                     preferred_element_type=jnp.float32)
        o_ref[...] = h * dinv_ref[...]
        ob_ref[...] = (hs * dinv_ref[...]).astype(jnp.bfloat16)

    return pl.pallas_call(
        body, out_shape=(jax.ShapeDtypeStruct((N, D), jnp.float32),
                         jax.ShapeDtypeStruct((N, D), jnp.bfloat16)))(
                             x, w, ws, dinv)


def _tc_post_mm(acc, g, dinv, b, w, ws):
    def body(acc_ref, g_ref, dinv_ref, b_ref, w_ref, ws_ref, o_ref, ob_ref):
        agg = acc_ref[0, :N, :] + acc_ref[1, :N, :] + g_ref[...]
        x1 = jnp.maximum(dinv_ref[...] * agg + b_ref[...], 0.0)
        o_ref[...] = jnp.dot(
            x1, w_ref[...], preferred_element_type=jnp.float32) * dinv_ref[...]
        ob_ref[...] = (jnp.dot(
            x1, ws_ref[...],
            preferred_element_type=jnp.float32) * dinv_ref[...]).astype(
                jnp.bfloat16)

    return pl.pallas_call(
        body, out_shape=(jax.ShapeDtypeStruct((N, D), jnp.float32),
                         jax.ShapeDtypeStruct((N, D), jnp.bfloat16)))(
                             acc, g, dinv, b, w, ws)


def _tc_post_final(acc, g, dinv, b):
    def body(acc_ref, g_ref, dinv_ref, b_ref, o_ref):
        agg = acc_ref[0, :N, :] + acc_ref[1, :N, :] + g_ref[...]
        o_ref[...] = jnp.maximum(dinv_ref[...] * agg + b_ref[...], 0.0)

    return pl.pallas_call(
        body, out_shape=jax.ShapeDtypeStruct((N, D), jnp.float32))(
            acc, g, dinv, b)


@jax.jit
def kernel(x, edge_index, edge_weight, W1, b1, W2, b2):
    src = edge_index[0].astype(jnp.int32)
    dst = edge_index[1].astype(jnp.int32)
    ew = edge_weight.astype(jnp.float32)
    pad = EPAD - E
    src_p = jnp.pad(src, (0, pad))
    dst_p = jnp.pad(dst, (0, pad))
    ew_p = jnp.pad(ew, (0, pad))
    src4 = src_p.reshape(TOTG, G, K)
    dst4 = dst_p.reshape(TOTG, G, K)
    ew4 = ew_p.reshape(TOTG, G * K)
    dst3 = dst_p.reshape(NW, CWD, KD)
    ew3 = ew_p.reshape(NW, CWD, KD)
    zdeg = jnp.zeros((NPAD,), jnp.float32)
    zrow = jnp.zeros((NPAD, D), jnp.float32)
    perm = jnp.asarray(_PERM, dtype=jnp.int32)
    W1s = jnp.take(W1, perm, axis=1)
    W2s = jnp.take(W2, perm, axis=1)

    def as_words(gb):
        return lax.bitcast_convert_type(
            gb.reshape(N, D // 2, 2), jnp.int32)

    deg2 = _sc_deg(dst3, ew3, zdeg)
    dinv_row = _tc_dinv(deg2)
    dinv_col = dinv_row[0, :N][:, None]

    g1, g1b = _tc_mm_scale(x, W1, W1s, dinv_col)
    acc1 = _sc_edge(as_words(g1b), src4, dst4, ew4, zrow)
    g2, g2b = _tc_post_mm(acc1, g1, dinv_col, b1.reshape(1, D), W2, W2s)
    acc2 = _sc_edge(as_words(g2b), src4, dst4, ew4, zrow)
    return _tc_post_final(acc2, g2, dinv_col, b2.reshape(1, D))


# speedup vs baseline: 1.1661x; 1.0892x over previous
import functools

import jax
import jax.numpy as jnp
from jax import lax
from jax.experimental import pallas as pl
from jax.experimental.pallas import tpu as pltpu
from jax.experimental.pallas import tpu_sc as plsc

N = 10000
E = 320000
D = 128
NC = 2
NS = 16
L = 16
NW = NC * NS
K = 64
G = 16
NG0 = 10
NG1 = 10
NG = NG0 + NG1
CWD = 80
KD = 128
TOTG = NS * NG
EPAD = TOTG * G * K
NPAD = 10240
RPS = NPAD // NS

def _mk_perm():
    pi = [0] * D
    for m in range(D // 32):
        for i in range(16):
            pi[32 * m + i] = 32 * m + 2 * i
            pi[32 * m + 16 + i] = 32 * m + 2 * i + 1
    inv = [0] * D
    for t, q in enumerate(pi):
        inv[q] = t
    return tuple(inv)

_PERM = _mk_perm()

_mesh = plsc.VectorSubcoreMesh(core_axis_name="c", subcore_axis_name="s")
_sc_params = pltpu.CompilerParams(needs_layout_passes=False,
                                  use_tc_tiling_on_sc=False)


@functools.partial(
    pl.kernel,
    out_type=jax.ShapeDtypeStruct((NC, NPAD), jnp.float32),
    mesh=_mesh,
    compiler_params=_sc_params,
    scratch_types=[
        pltpu.VMEM((CWD, KD), jnp.int32),
        pltpu.VMEM((CWD, KD), jnp.float32),
        pltpu.VMEM_SHARED((NPAD,), jnp.float32),
    ],
)
def _sc_deg(dst_hbm, ew_hbm, zdeg_hbm, deg_out, dst_v, ew_v, deg_sh):
    c = lax.axis_index("c")
    s = lax.axis_index("s")
    wid = s * NC + c
    pltpu.sync_copy(zdeg_hbm.at[pl.ds(s * RPS, RPS)],
                    deg_sh.at[pl.ds(s * RPS, RPS)])
    pltpu.sync_copy(dst_hbm.at[wid], dst_v)
    pltpu.sync_copy(ew_hbm.at[wid], ew_v)
    plsc.subcore_barrier()

    @pl.loop(0, CWD)
    def _(t):
        pltpu.sync_copy(ew_v.at[t], deg_sh.at[dst_v.at[t]], add=True)

    plsc.subcore_barrier()

    @pl.when(s == 0)
    def _():
        pltpu.sync_copy(deg_sh, deg_out.at[c])


@functools.partial(
    pl.kernel,
    out_type=jax.ShapeDtypeStruct((NC, NPAD, D), jnp.float32),
    mesh=_mesh,
    compiler_params=_sc_params,
    scratch_types=[
        pltpu.VMEM((2, G, K), jnp.int32),
        pltpu.VMEM((2, G, K), jnp.int32),
        pltpu.VMEM((2, G * K), jnp.float32),
        pltpu.VMEM((K, D // 2), jnp.int32),
        pltpu.VMEM((K, D // 2), jnp.int32),
        pltpu.VMEM((K, D // 2), jnp.int32),
        pltpu.VMEM((K, D // 2), jnp.int32),
        pltpu.VMEM((K, D), jnp.float32),
        pltpu.VMEM((K, D), jnp.float32),
        pltpu.VMEM_SHARED((NPAD, D), jnp.float32),
        pltpu.SemaphoreType.DMA,
        pltpu.SemaphoreType.DMA,
        pltpu.SemaphoreType.DMA,
        pltpu.SemaphoreType.DMA,
        pltpu.SemaphoreType.DMA,
        pltpu.SemaphoreType.DMA,
        pltpu.SemaphoreType.DMA,
    ],
)
def _sc_edge(g_hbm, src_hbm, dst_hbm, ew_hbm, zrow_hbm, acc_out,
             src_v, dst_v, ew_v, rb0, rb1, rb2, rb3, rc0, rc1, acc_sh,
             sg0, sg1, sg2, sg3, ss0, ss1, si):
    c = lax.axis_index("c")
    s = lax.axis_index("s")
    ng = jnp.where(c == 0, NG0, NG1)
    gb = jnp.where(c == 0, s * NG0, NS * NG0 + s * NG1)
    rb = [rb0, rb1, rb2, rb3]
    rc = [rc0, rc1]
    sg = [sg0, sg1, sg2, sg3]
    ss = [ss0, ss1]

    pltpu.sync_copy(zrow_hbm.at[pl.ds(s * RPS, RPS)],
                    acc_sh.at[pl.ds(s * RPS, RPS)])

    def load_group(gi, p):
        pltpu.make_async_copy(src_hbm.at[gb + gi], src_v.at[p], si).start()
        pltpu.make_async_copy(dst_hbm.at[gb + gi], dst_v.at[p], si).start()
        pltpu.make_async_copy(ew_hbm.at[gb + gi], ew_v.at[p], si).start()

    def wait_group(p):
        pltpu.make_async_copy(src_hbm.at[gb], src_v.at[p], si).wait()
        pltpu.make_async_copy(dst_hbm.at[gb], dst_v.at[p], si).wait()
        pltpu.make_async_copy(ew_hbm.at[gb], ew_v.at[p], si).wait()

    def start_gather(p, t, i):
        pltpu.make_async_copy(g_hbm.at[src_v.at[p, t]], rb[i], sg[i]).start()

    def wait_gather(p, i):
        pltpu.make_async_copy(g_hbm.at[src_v.at[p, 0]], rb[i], sg[i]).wait()

    def start_scatter(p, t, j):
        pltpu.make_async_copy(
            rc[j], acc_sh.at[dst_v.at[p, t]], ss[j]).start(add=True)

    def wait_scatter(p, j):
        pltpu.make_async_copy(rc[j], acc_sh.at[dst_v.at[p, 0]], ss[j]).wait()

    def scale(p, t, i, j):
        @plsc.parallel_loop(0, K, unroll=2)
        def _(k):
            w = plsc.load_gather(
                ew_v, [jnp.full((L,), p, jnp.int32),
                       jnp.full((L,), t * K + k, jnp.int32)])
            for m in range(D // 32):
                v32 = rb[i][k, pl.ds(16 * m, 16)]
                v = plsc.bitcast(v32, jnp.bfloat16)
                a, b = plsc.unpack(v, format=plsc.PackFormat.INTERLEAVED)
                rc[j][k, pl.ds(32 * m, L)] = a * w
                rc[j][k, pl.ds(32 * m + L, L)] = b * w

    @pl.when(ng > 0)
    def _():
        load_group(0, 0)

    plsc.subcore_barrier()

    @pl.when(ng > 0)
    def _():
        wait_group(0)
        for i in range(4):
            start_gather(0, i, i)

    @pl.loop(0, NG)
    def _(gi):
        @pl.when(gi < ng)
        def _():
            p = lax.rem(gi, 2)

            @pl.loop(0, G // 4)
            def _(q):
                for i in range(4):
                    t = q * 4 + i
                    j = i % 2
                    wait_gather(p, i)
                    if i < 2:
                        @pl.when(jnp.logical_or(gi > 0, q > 0))
                        def _():
                            wait_scatter(p, j)
                    else:
                        wait_scatter(p, j)
                    if i == 0:
                        @pl.when(jnp.logical_and(q == 1, gi + 1 < ng))
                        def _():
                            load_group(gi + 1, 1 - p)
                    scale(p, t, i, j)
                    start_scatter(p, t, j)

                    @pl.when(q < G // 4 - 1)
                    def _():
                        start_gather(p, t + 4, i)

                    @pl.when(jnp.logical_and(q == G // 4 - 1, gi + 1 < ng))
                    def _():
                        if i == 0:
                            wait_group(1 - p)
                        start_gather(1 - p, i, i)

    @pl.when(ng > 0)
    def _():
        wait_scatter(0, 0)
        wait_scatter(0, 1)

    plsc.subcore_barrier()
    pltpu.sync_copy(acc_sh.at[pl.ds(s * RPS, RPS)],
                    acc_out.at[c, pl.ds(s * RPS, RPS)])


def _tc_dinv(deg2):
    def body(deg_ref, out_ref):
        d = deg_ref[0:1, :] + deg_ref[1:2, :] + 1.0
        out_ref[...] = jnp.where(d > 0, lax.rsqrt(d), 0.0)

    return pl.pallas_call(
        body, out_shape=jax.ShapeDtypeStruct((1, NPAD), jnp.float32))(deg2)


def _tc_mm_scale(x, w, ws, dinv):
    def body(x_ref, w_ref, ws_ref, dinv_ref, o_ref, ob_ref):
        h = jnp.dot(x_ref[...], w_ref[...], preferred_element_type=jnp.float32)
        hs = jnp.dot(x_ref[...], ws_ref[...],
                     preferred_element_type=jnp.float32)
        o_ref[...] = h * dinv_ref[...]
        ob_ref[...] = (hs * dinv_ref[...]).astype(jnp.bfloat16)

    return pl.pallas_call(
        body, out_shape=(jax.ShapeDtypeStruct((N, D), jnp.float32),
                         jax.ShapeDtypeStruct((N, D), jnp.bfloat16)))(
                             x, w, ws, dinv)


def _tc_post_mm(acc, g, dinv, b, w, ws):
    def body(acc_ref, g_ref, dinv_ref, b_ref, w_ref, ws_ref, o_ref, ob_ref):
        agg = acc_ref[0, :N, :] + acc_ref[1, :N, :] + g_ref[...]
        x1 = jnp.maximum(dinv_ref[...] * agg + b_ref[...], 0.0)
        o_ref[...] = jnp.dot(
            x1, w_ref[...], preferred_element_type=jnp.float32) * dinv_ref[...]
        ob_ref[...] = (jnp.dot(
            x1, ws_ref[...],
            preferred_element_type=jnp.float32) * dinv_ref[...]).astype(
                jnp.bfloat16)

    return pl.pallas_call(
        body, out_shape=(jax.ShapeDtypeStruct((N, D), jnp.float32),
                         jax.ShapeDtypeStruct((N, D), jnp.bfloat16)))(
                             acc, g, dinv, b, w, ws)


def _tc_post_final(acc, g, dinv, b):
    def body(acc_ref, g_ref, dinv_ref, b_ref, o_ref):
        agg = acc_ref[0, :N, :] + acc_ref[1, :N, :] + g_ref[...]
        o_ref[...] = jnp.maximum(dinv_ref[...] * agg + b_ref[...], 0.0)

    return pl.pallas_call(
        body, out_shape=jax.ShapeDtypeStruct((N, D), jnp.float32))(
            acc, g, dinv, b)


@jax.jit
def kernel(x, edge_index, edge_weight, W1, b1, W2, b2):
    src = edge_index[0].astype(jnp.int32)
    dst = edge_index[1].astype(jnp.int32)
    ew = edge_weight.astype(jnp.float32)
    pad = EPAD - E
    src_p = jnp.pad(src, (0, pad))
    dst_p = jnp.pad(dst, (0, pad))
    ew_p = jnp.pad(ew, (0, pad))
    src4 = src_p.reshape(TOTG, G, K)
    dst4 = dst_p.reshape(TOTG, G, K)
    ew4 = ew_p.reshape(TOTG, G * K)
    dst3 = dst_p.reshape(NW, CWD, KD)
    ew3 = ew_p.reshape(NW, CWD, KD)
    zdeg = jnp.zeros((NPAD,), jnp.float32)
    zrow = jnp.zeros((NPAD, D), jnp.float32)
    perm = jnp.asarray(_PERM, dtype=jnp.int32)
    W1s = jnp.take(W1, perm, axis=1)
    W2s = jnp.take(W2, perm, axis=1)

    def as_words(gb):
        return lax.bitcast_convert_type(
            gb.reshape(N, D // 2, 2), jnp.int32)

    deg2 = _sc_deg(dst3, ew3, zdeg)
    dinv_row = _tc_dinv(deg2)
    dinv_col = dinv_row[0, :N][:, None]

    g1, g1b = _tc_mm_scale(x, W1, W1s, dinv_col)
    acc1 = _sc_edge(as_words(g1b), src4, dst4, ew4, zrow)
    g2, g2b = _tc_post_mm(acc1, g1, dinv_col, b1.reshape(1, D), W2, W2s)
    acc2 = _sc_edge(as_words(g2b), src4, dst4, ew4, zrow)
    return _tc_post_final(acc2, g2, dinv_col, b2.reshape(1, D))
